# baseline (speedup 1.0000x reference)
import jax
import jax.numpy as jnp
from jax.experimental import pallas as pl
from jax.experimental.pallas import tpu as pltpu

_M = 512


def _tile_body(ids_ref, x_ref, w_ref, tb_ref, out_ref):
    x = x_ref[...].astype(jnp.bfloat16)
    ids = ids_ref[0, 0, :]
    n_sub = tb_ref.shape[0]
    in_f = x.shape[1]
    idsb = jnp.broadcast_to(ids[:, None], (x.shape[0], in_f))
    zero = jnp.zeros_like(x)
    xp = jnp.concatenate(
        [jnp.where(idsb == s, x, zero) for s in range(n_sub)], axis=1)
    acc = jnp.dot(xp, w_ref[...], preferred_element_type=jnp.float32)
    oh = (ids[:, None] == jax.lax.broadcasted_iota(jnp.int32, (1, n_sub), 1)
          ).astype(jnp.float32)
    out_ref[...] = acc + jnp.dot(oh, tb_ref[...],
                                 preferred_element_type=jnp.float32)


def kernel(feat, subdet_id, proj_w, proj_b, type_table):
    n, in_f = feat.shape
    n_sub, embed = type_table.shape
    ids3 = subdet_id.reshape(n // _M, 1, _M)
    w2 = proj_w.reshape(n_sub * in_f, embed).astype(jnp.bfloat16)
    tb = proj_b + type_table
    return pl.pallas_call(
        _tile_body,
        grid=(n // _M,),
        in_specs=[
            pl.BlockSpec((1, 1, _M), lambda i: (i, 0, 0)),
            pl.BlockSpec((_M, in_f), lambda i: (i, 0)),
            pl.BlockSpec((n_sub * in_f, embed), lambda i: (0, 0)),
            pl.BlockSpec((n_sub, embed), lambda i: (0, 0)),
        ],
        out_specs=pl.BlockSpec((_M, embed), lambda i: (i, 0)),
        out_shape=jax.ShapeDtypeStruct((n, embed), jnp.float32),
        compiler_params=pltpu.CompilerParams(
            dimension_semantics=("parallel",)),
    )(ids3, feat, w2, tb)


# speedup vs baseline: 1.4489x; 1.4489x over previous
import jax
import jax.numpy as jnp
from jax.experimental import pallas as pl
from jax.experimental.pallas import tpu as pltpu

_M = 2048


def _tile_body(ids_ref, x_ref, w_ref, tb_ref, out_ref):
    x = x_ref[...].astype(jnp.bfloat16)
    ids = ids_ref[0, 0, :]
    n_sub = tb_ref.shape[0]
    in_f = x.shape[1]
    idsb = jnp.broadcast_to(ids[:, None], (x.shape[0], in_f))
    zero = jnp.zeros_like(x)
    xp = jnp.concatenate(
        [jnp.where(idsb == s, x, zero) for s in range(n_sub)], axis=1)
    acc = jnp.dot(xp, w_ref[...], preferred_element_type=jnp.float32)
    oh = (ids[:, None] == jax.lax.broadcasted_iota(jnp.int32, (1, n_sub), 1)
          ).astype(jnp.float32)
    out_ref[...] = acc + jnp.dot(oh, tb_ref[...],
                                 preferred_element_type=jnp.float32)


def kernel(feat, subdet_id, proj_w, proj_b, type_table):
    n, in_f = feat.shape
    n_sub, embed = type_table.shape
    ids3 = subdet_id.reshape(n // _M, 1, _M)
    w2 = proj_w.reshape(n_sub * in_f, embed).astype(jnp.bfloat16)
    tb = proj_b + type_table
    return pl.pallas_call(
        _tile_body,
        grid=(n // _M,),
        in_specs=[
            pl.BlockSpec((1, 1, _M), lambda i: (i, 0, 0)),
            pl.BlockSpec((_M, in_f), lambda i: (i, 0)),
            pl.BlockSpec((n_sub * in_f, embed), lambda i: (0, 0)),
            pl.BlockSpec((n_sub, embed), lambda i: (0, 0)),
        ],
        out_specs=pl.BlockSpec((_M, embed), lambda i: (i, 0)),
        out_shape=jax.ShapeDtypeStruct((n, embed), jnp.float32),
        compiler_params=pltpu.CompilerParams(
            dimension_semantics=("parallel",)),
    )(ids3, feat, w2, tb)


# speedup vs baseline: 1.4729x; 1.0166x over previous
import jax
import jax.numpy as jnp
from jax.experimental import pallas as pl
from jax.experimental.pallas import tpu as pltpu

_M = 4096


def _tile_body(ids_ref, x_ref, w_ref, tb_ref, out_ref):
    x = x_ref[...].astype(jnp.bfloat16)
    ids = ids_ref[0, 0, :]
    n_sub = tb_ref.shape[0]
    in_f = x.shape[1]
    idsb = jnp.broadcast_to(ids[:, None], (x.shape[0], in_f))
    zero = jnp.zeros_like(x)
    xp = jnp.concatenate(
        [jnp.where(idsb == s, x, zero) for s in range(n_sub)], axis=1)
    acc = jnp.dot(xp, w_ref[...], preferred_element_type=jnp.float32)
    oh = (ids[:, None] == jax.lax.broadcasted_iota(jnp.int32, (1, n_sub), 1)
          ).astype(jnp.float32)
    out_ref[...] = acc + jnp.dot(oh, tb_ref[...],
                                 preferred_element_type=jnp.float32)


def kernel(feat, subdet_id, proj_w, proj_b, type_table):
    n, in_f = feat.shape
    n_sub, embed = type_table.shape
    ids3 = subdet_id.reshape(n // _M, 1, _M)
    w2 = proj_w.reshape(n_sub * in_f, embed).astype(jnp.bfloat16)
    tb = proj_b + type_table
    return pl.pallas_call(
        _tile_body,
        grid=(n // _M,),
        in_specs=[
            pl.BlockSpec((1, 1, _M), lambda i: (i, 0, 0)),
            pl.BlockSpec((_M, in_f), lambda i: (i, 0)),
            pl.BlockSpec((n_sub * in_f, embed), lambda i: (0, 0)),
            pl.BlockSpec((n_sub, embed), lambda i: (0, 0)),
        ],
        out_specs=pl.BlockSpec((_M, embed), lambda i: (i, 0)),
        out_shape=jax.ShapeDtypeStruct((n, embed), jnp.float32),
        compiler_params=pltpu.CompilerParams(
            dimension_semantics=("parallel",)),
    )(ids3, feat, w2, tb)


# speedup vs baseline: 1.5159x; 1.0292x over previous
import jax
import jax.numpy as jnp
from jax.experimental import pallas as pl
from jax.experimental.pallas import tpu as pltpu

_M = 4096


def _tile_body(ids_ref, x_ref, w_ref, out_ref):
    x = x_ref[...].astype(jnp.bfloat16)
    ids = ids_ref[0, 0, :]
    n_sub = 8
    in_f = x.shape[1]
    ids16 = ids.astype(jnp.int16)
    idsb = jnp.broadcast_to(ids16[:, None], (x.shape[0], in_f))
    zero = jnp.zeros_like(x)
    oh = (ids16[:, None] == jax.lax.broadcasted_iota(jnp.int16, (1, n_sub), 1)
          ).astype(jnp.bfloat16)
    xp = jnp.concatenate(
        [jnp.where(idsb == jnp.int16(s), x, zero) for s in range(n_sub)]
        + [oh], axis=1)
    out_ref[...] = jnp.dot(xp, w_ref[...], preferred_element_type=jnp.float32)


def kernel(feat, subdet_id, proj_w, proj_b, type_table):
    n, in_f = feat.shape
    n_sub, embed = type_table.shape
    ids3 = subdet_id.reshape(n // _M, 1, _M)
    w2 = proj_w.reshape(n_sub * in_f, embed)
    tb = proj_b + type_table
    w3 = jnp.concatenate([w2, tb], axis=0).astype(jnp.bfloat16)
    return pl.pallas_call(
        _tile_body,
        grid=(n // _M,),
        in_specs=[
            pl.BlockSpec((1, 1, _M), lambda i: (i, 0, 0)),
            pl.BlockSpec((_M, in_f), lambda i: (i, 0)),
            pl.BlockSpec((n_sub * in_f + n_sub, embed), lambda i: (0, 0)),
        ],
        out_specs=pl.BlockSpec((_M, embed), lambda i: (i, 0)),
        out_shape=jax.ShapeDtypeStruct((n, embed), jnp.float32),
        compiler_params=pltpu.CompilerParams(
            dimension_semantics=("parallel",)),
    )(ids3, feat, w3)
